# Initial kernel scaffold; baseline (speedup 1.0000x reference)
import jax
import jax.numpy as jnp
from jax.experimental import pallas as pl


def kernel(x, edge_index, batch, W1, b1, W2, b2, Wfc, bfc):
    raise NotImplementedError("write your pallas kernel here")


# speedup vs baseline: 19.9820x; 19.9820x over previous
import functools

import jax
import jax.numpy as jnp
from jax import lax
from jax.experimental import pallas as pl
from jax.experimental.pallas import tpu as pltpu
from jax.experimental.pallas import tpu_sc as plsc

N = 10000
E = 320000
M = 10000
DI = 128
DH = 128
DO = 10
G = 64

NP = 10240
RPT = NP // 16
K = 128
EP = 327680
NB = 2

CH = 80
PPT = CH * K

DPT = EP // 32
HB = 12288
CPT = HB // 32


def _sc_pass_body(src_hbm, gidx_hbm, sidx_hbm, z_hbm, out_hbm,
                  gidx_v, rows_v, sq0, sq1, acc_sh, gsem, ssem):
    cid = lax.axis_index("c")
    sid = lax.axis_index("s")
    wid = sid * 2 + cid
    sq = (sq0, sq1)

    pltpu.sync_copy(gidx_hbm.at[pl.ds(wid * PPT, PPT)], gidx_v)

    base = sid * RPT
    for k in range(RPT // K):
        pltpu.sync_copy(z_hbm, acc_sh.at[pl.ds(base + k * K, K)])
    plsc.subcore_barrier()

    def g_start(c, b):
        pltpu.async_copy(src_hbm.at[gidx_v.at[pl.ds(c * K, K)]],
                         rows_v.at[b], gsem.at[b])

    def g_wait(c, b):
        pltpu.make_async_copy(src_hbm.at[gidx_v.at[pl.ds(c * K, K)]],
                              rows_v.at[b], gsem.at[b]).wait()

    def s_load(c, b):
        pltpu.async_copy(sidx_hbm.at[pl.ds(wid * PPT + c * K, K)],
                         sq[b], ssem.at[b])

    def s_wait(c, b):
        pltpu.make_async_copy(sidx_hbm.at[pl.ds(wid * PPT + c * K, K)],
                              sq[b], ssem.at[b]).wait()

    for b in range(NB):
        g_start(b, b)
        s_load(b, b)

    def body(i, carry):
        for b in range(NB):
            c = i * NB + b
            g_wait(c, b)
            s_wait(c, b)
            pltpu.sync_copy(rows_v.at[b], acc_sh.at[sq[b]], add=True)
            c2 = jnp.minimum(c + NB, CH - 1)
            g_start(c2, b)
            s_load(c2, b)
        return carry

    lax.fori_loop(0, CH // NB, body, 0)
    for b in range(NB):
        g_wait(CH - 1, b)
        s_wait(CH - 1, b)

    plsc.subcore_barrier()
    for k in range(RPT // K):
        pltpu.sync_copy(acc_sh.at[pl.ds(base + k * K, K)],
                        out_hbm.at[cid, pl.ds(base + k * K, K)])


_sc_pass = functools.partial(
    pl.kernel,
    out_type=jax.ShapeDtypeStruct((2, NP, DH), jnp.float32),
    mesh=plsc.VectorSubcoreMesh(core_axis_name="c", subcore_axis_name="s",
                                num_cores=2, num_subcores=16),
    scratch_types=[
        pltpu.VMEM((PPT,), jnp.int32),
        pltpu.VMEM((NB, K, DH), jnp.float32),
        pltpu.VMEM((K,), jnp.int32),
        pltpu.VMEM((K,), jnp.int32),
        pltpu.VMEM_SHARED((NP, DH), jnp.float32),
        pltpu.SemaphoreType.DMA((NB,)),
        pltpu.SemaphoreType.DMA((NB,)),
    ],
    compiler_params=pltpu.CompilerParams(needs_layout_passes=False),
)(_sc_pass_body)


def _sc_deg_body(nidx_hbm, hidx_hbm, dcnt_hbm, bcnt_hbm,
                 nidx_v, hidx_v, hist_n, hist_m, red_v, ob_v, sh_n, sh_m):
    cid = lax.axis_index("c")
    sid = lax.axis_index("s")
    wid = sid * 2 + cid

    pltpu.sync_copy(nidx_hbm.at[pl.ds(wid * DPT, DPT)], nidx_v)
    pltpu.sync_copy(hidx_hbm.at[pl.ds(wid * DPT, DPT)], hidx_v)

    zero16 = jnp.zeros((16,), jnp.float32)

    def zbody(i, carry):
        hist_n[pl.ds(i * 16, 16)] = zero16
        hist_m[pl.ds(i * 16, 16)] = zero16
        return carry

    lax.fori_loop(0, HB // 16, zbody, 0)

    ones16 = jnp.ones((16,), jnp.float32)

    def hbody(i, carry):
        plsc.addupdate_scatter(hist_n, [nidx_v[pl.ds(i * 16, 16)]], ones16)
        plsc.addupdate_scatter(hist_m, [hidx_v[pl.ds(i * 16, 16)]], ones16)
        return carry

    lax.fori_loop(0, DPT // 16, hbody, 0)

    pltpu.sync_copy(hist_n, sh_n.at[pl.ds(sid * HB, HB)])
    pltpu.sync_copy(hist_m, sh_m.at[pl.ds(sid * HB, HB)])
    plsc.subcore_barrier()

    col = sid * CPT
    for sh, out_hbm in ((sh_n, dcnt_hbm), (sh_m, bcnt_hbm)):
        for r in range(16):
            pltpu.sync_copy(sh.at[pl.ds(r * HB + col, CPT)],
                            red_v.at[pl.ds(r * CPT, CPT)])

        def rbody(k, carry):
            acc = red_v[pl.ds(k * 16, 16)]
            for r in range(1, 16):
                acc = acc + red_v[pl.ds(r * CPT + k * 16, 16)]
            ob_v[pl.ds(k * 16, 16)] = acc
            return carry

        lax.fori_loop(0, CPT // 16, rbody, 0)
        pltpu.sync_copy(ob_v, out_hbm.at[pl.ds(cid * HB + col, CPT)])


_sc_deg = functools.partial(
    pl.kernel,
    out_type=(jax.ShapeDtypeStruct((2 * HB,), jnp.float32),
              jax.ShapeDtypeStruct((2 * HB,), jnp.float32)),
    mesh=plsc.VectorSubcoreMesh(core_axis_name="c", subcore_axis_name="s",
                                num_cores=2, num_subcores=16),
    scratch_types=[
        pltpu.VMEM((DPT,), jnp.int32),
        pltpu.VMEM((DPT,), jnp.int32),
        pltpu.VMEM((HB,), jnp.float32),
        pltpu.VMEM((HB,), jnp.float32),
        pltpu.VMEM((16 * CPT,), jnp.float32),
        pltpu.VMEM((CPT,), jnp.float32),
        pltpu.VMEM_SHARED((16 * HB,), jnp.float32),
        pltpu.VMEM_SHARED((16 * HB,), jnp.float32),
    ],
    compiler_params=pltpu.CompilerParams(needs_layout_passes=False),
)(_sc_deg_body)


def _mm_body(x_ref, w_ref, o_ref):
    o_ref[...] = jnp.dot(x_ref[...], w_ref[...],
                         preferred_element_type=jnp.float32)


def _tc_mm(x, w):
    return pl.pallas_call(
        _mm_body,
        out_shape=jax.ShapeDtypeStruct((x.shape[0], w.shape[1]), jnp.float32),
    )(x, w)


def _inv_counts(cnt_ref):
    c = cnt_ref[...]
    tot = c[:, 0:1] + c[:, 1:2]
    return jnp.where(tot > 0, 1.0 / tot, 0.0)[:NP]


def _scale_body(s_ref, cnt_ref, o_ref):
    o_ref[...] = (s_ref[0] + s_ref[1]) * _inv_counts(cnt_ref)


def _tc_scale(s, cnt_t):
    return pl.pallas_call(
        _scale_body,
        out_shape=jax.ShapeDtypeStruct((NP, DH), jnp.float32),
    )(s, cnt_t)


def _relu_mm_body(u_ref, cnt_ref, b_ref, w_ref, o_ref):
    h = jnp.maximum((u_ref[0] + u_ref[1]) * _inv_counts(cnt_ref)
                    + b_ref[...], 0.0)
    rows = lax.broadcasted_iota(jnp.int32, (NP, 1), 0)
    h = jnp.where(rows < N, h, 0.0)
    o_ref[...] = jnp.dot(h, w_ref[...], preferred_element_type=jnp.float32)


def _tc_relu_mm(u, cnt_t, b, w):
    return pl.pallas_call(
        _relu_mm_body,
        out_shape=jax.ShapeDtypeStruct((NP, DH), jnp.float32),
    )(u, cnt_t, b, w)


def _final_body(u_ref, cnt_ref, b_ref, batch_ref, wfc_ref, bfc_ref, o_ref):
    h = jnp.maximum((u_ref[0] + u_ref[1]) * _inv_counts(cnt_ref)
                    + b_ref[...], 0.0)
    rows = lax.broadcasted_iota(jnp.int32, (NP, 1), 0)
    h = jnp.where(rows < N, h, 0.0)

    bb = batch_ref[...]
    gids = lax.broadcasted_iota(jnp.int32, (G, NP), 0)
    oh = jnp.where(gids == bb, 1.0, 0.0)
    pooled = jnp.dot(oh, h, preferred_element_type=jnp.float32)
    counts = jnp.sum(oh, axis=1, keepdims=True)
    gm = pooled / jnp.maximum(counts, 1.0)
    logits = jnp.dot(gm, wfc_ref[...],
                     preferred_element_type=jnp.float32) + bfc_ref[...]
    mx = jnp.max(logits, axis=1, keepdims=True)
    shl = logits - mx
    o_ref[...] = shl - jnp.log(jnp.sum(jnp.exp(shl), axis=1, keepdims=True))


def _tc_final(u, cnt_t, b, batch_p, wfc, bfc):
    return pl.pallas_call(
        _final_body,
        out_shape=jax.ShapeDtypeStruct((G, DO), jnp.float32),
    )(u, cnt_t, b, batch_p, wfc, bfc)


def kernel(x, edge_index, batch, W1, b1, W2, b2, Wfc, bfc):
    x_pad = jnp.zeros((NP, DI), jnp.float32).at[:N, :].set(x)
    pad = (jnp.arange(EP - E, dtype=jnp.int32) % (NP - N)) + N
    nidx = jnp.concatenate([edge_index[0], pad])
    hidx = jnp.concatenate([edge_index[1], pad])
    z128 = jnp.zeros((K, DH), jnp.float32)
    batch_p = jnp.concatenate(
        [batch, jnp.full((NP - N,), G, jnp.int32)]).reshape(1, NP)

    dcnt, bcnt = _sc_deg(nidx, hidx)
    dcnt_t = dcnt.reshape(2, HB).T
    bcnt_t = bcnt.reshape(2, HB).T

    t1 = _tc_mm(x_pad, W1)
    s1 = _sc_pass(t1, nidx, hidx, z128)
    he1 = _tc_scale(s1, bcnt_t)
    u1 = _sc_pass(he1, hidx, nidx, z128)
    t2 = _tc_relu_mm(u1, dcnt_t, b1.reshape(1, DH), W2)
    s2 = _sc_pass(t2, nidx, hidx, z128)
    he2 = _tc_scale(s2, bcnt_t)
    u2 = _sc_pass(he2, hidx, nidx, z128)
    return _tc_final(u2, dcnt_t, b2.reshape(1, DH), batch_p, Wfc,
                     bfc.reshape(1, DO))
